# Initial kernel scaffold; baseline (speedup 1.0000x reference)
import jax
import jax.numpy as jnp
from jax.experimental import pallas as pl


def kernel(x, edge_index, batch, W1, b1, W2, b2, W3, b3, W4, b4):
    raise NotImplementedError("write your pallas kernel here")


# speedup vs baseline: 20.8356x; 20.8356x over previous
import functools

import jax
import jax.numpy as jnp
from jax import lax
from jax.experimental import pallas as pl
from jax.experimental.pallas import tpu as pltpu
from jax.experimental.pallas import tpu_sc as plsc

NC = 2
NS = 16
NW = NC * NS
K = 128
TRASH = 8


def _make_prop_colsplit(n_out, n_pad, nch, dh):
    mesh = plsc.VectorSubcoreMesh(core_axis_name="c", subcore_axis_name="s")

    @functools.partial(
        pl.kernel,
        out_type=jax.ShapeDtypeStruct((NC, n_out, dh), jnp.float32),
        mesh=mesh,
        compiler_params=pltpu.CompilerParams(use_tc_tiling_on_sc=False),
        scratch_types=[
            pltpu.VMEM((nch, K), jnp.int32),
            pltpu.VMEM((nch, K), jnp.int32),
            pltpu.VMEM((2, K, dh), jnp.float32),
            pltpu.VMEM_SHARED((n_pad, dh), jnp.float32),
            pltpu.SemaphoreType.DMA((2,)),
        ],
    )
    def prop(t_hbm, src_hbm, dst_hbm, out_hbm, src_v, dst_v, buf, acc, sem):
        c = lax.axis_index("c")
        s = lax.axis_index("s")
        pltpu.sync_copy(src_hbm.at[s], src_v)
        pltpu.sync_copy(dst_hbm.at[s], dst_v)

        @pl.when(s == 0)
        def _():
            pltpu.sync_copy(t_hbm.at[c], acc)

        plsc.subcore_barrier()

        pltpu.async_copy(t_hbm.at[c].at[src_v.at[0]], buf.at[0], sem.at[0])

        def body(j, carry):
            slot = lax.rem(j, 2)
            nxt = lax.rem(j + 1, 2)

            @pl.when(j + 1 < nch)
            def _():
                pltpu.async_copy(t_hbm.at[c].at[src_v.at[j + 1]],
                                 buf.at[nxt], sem.at[nxt])

            pltpu.make_async_copy(t_hbm.at[c].at[src_v.at[j]], buf.at[slot],
                                  sem.at[slot]).wait()
            pltpu.sync_copy(buf.at[slot], acc.at[dst_v.at[j]], add=True)
            return carry

        lax.fori_loop(0, nch, body, 0)
        plsc.subcore_barrier()

        @pl.when(s == 0)
        def _():
            pltpu.sync_copy(acc.at[pl.ds(0, n_out)], out_hbm.at[c])

    return prop


def _make_prop_edgesplit(n_out, n_pad, nch, dw):
    mesh = plsc.VectorSubcoreMesh(core_axis_name="c", subcore_axis_name="s")

    @functools.partial(
        pl.kernel,
        out_type=jax.ShapeDtypeStruct((NC, n_out, dw), jnp.float32),
        mesh=mesh,
        compiler_params=pltpu.CompilerParams(use_tc_tiling_on_sc=False),
        scratch_types=[
            pltpu.VMEM((nch, K), jnp.int32),
            pltpu.VMEM((nch, K), jnp.int32),
            pltpu.VMEM((2, K, dw), jnp.float32),
            pltpu.VMEM_SHARED((n_pad, dw), jnp.float32),
            pltpu.SemaphoreType.DMA((2,)),
        ],
    )
    def prop(t_hbm, src_hbm, dst_hbm, zeros_hbm, out_hbm,
             src_v, dst_v, buf, acc, sem):
        c = lax.axis_index("c")
        s = lax.axis_index("s")
        pltpu.sync_copy(src_hbm.at[c].at[s], src_v)
        pltpu.sync_copy(dst_hbm.at[c].at[s], dst_v)

        @pl.when(s == 0)
        def _():
            @pl.when(c == 0)
            def _():
                pltpu.sync_copy(t_hbm, acc)

            @pl.when(c != 0)
            def _():
                pltpu.sync_copy(zeros_hbm, acc)

        plsc.subcore_barrier()

        pltpu.async_copy(t_hbm.at[src_v.at[0]], buf.at[0], sem.at[0])

        def body(j, carry):
            slot = lax.rem(j, 2)
            nxt = lax.rem(j + 1, 2)

            @pl.when(j + 1 < nch)
            def _():
                pltpu.async_copy(t_hbm.at[src_v.at[j + 1]], buf.at[nxt],
                                 sem.at[nxt])

            pltpu.make_async_copy(t_hbm.at[src_v.at[j]], buf.at[slot],
                                  sem.at[slot]).wait()
            pltpu.sync_copy(buf.at[slot], acc.at[dst_v.at[j]], add=True)
            return carry

        lax.fori_loop(0, nch, body, 0)
        plsc.subcore_barrier()

        @pl.when(s == 0)
        def _():
            pltpu.sync_copy(acc.at[pl.ds(0, n_out)], out_hbm.at[c])

    return prop


def _make_deg(n_out, n_pad, nch):
    mesh = plsc.VectorSubcoreMesh(core_axis_name="c", subcore_axis_name="s")

    @functools.partial(
        pl.kernel,
        out_type=jax.ShapeDtypeStruct((NC, n_out, 16), jnp.float32),
        mesh=mesh,
        compiler_params=pltpu.CompilerParams(use_tc_tiling_on_sc=False),
        scratch_types=[
            pltpu.VMEM((nch, K), jnp.int32),
            pltpu.VMEM((K, 16), jnp.float32),
            pltpu.VMEM_SHARED((n_pad, 16), jnp.float32),
        ],
    )
    def deg(dst_hbm, ones_hbm, zeros_hbm, out_hbm, dst_v, ones_v, acc):
        c = lax.axis_index("c")
        s = lax.axis_index("s")
        pltpu.sync_copy(dst_hbm.at[c].at[s], dst_v)
        pltpu.sync_copy(ones_hbm, ones_v)

        @pl.when(s == 0)
        def _():
            pltpu.sync_copy(zeros_hbm, acc)

        plsc.subcore_barrier()

        def body(j, carry):
            pltpu.sync_copy(ones_v, acc.at[dst_v.at[j]], add=True)
            return carry

        lax.fori_loop(0, nch, body, 0)
        plsc.subcore_barrier()

        @pl.when(s == 0)
        def _():
            pltpu.sync_copy(acc.at[pl.ds(0, n_out)], out_hbm.at[c])

    return deg


def _tc_call(body, out_shapes, *args):
    return pl.pallas_call(body, out_shape=out_shapes)(*args)


def kernel(x, edge_index, batch, W1, b1, W2, b2, W3, b3, W4, b4):
    n, d = x.shape
    e = edge_index.shape[1]
    dh = d // 2
    c_dim = W4.shape[1]
    g_dim = 16
    cp = 16
    n_pad = n + TRASH

    src = edge_index[0]
    dst = edge_index[1]

    epw2 = -(-e // NS)
    nch2 = -(-epw2 // K)
    pad2 = NS * nch2 * K - e
    pad2_src = jnp.full((pad2,), n, dtype=jnp.int32)
    pad2_dst = (jnp.arange(pad2, dtype=jnp.int32) % TRASH) + n
    srcs2 = jnp.concatenate([src, pad2_src]).reshape(NS, nch2, K)
    dsts2 = jnp.concatenate([dst, pad2_dst]).reshape(NS, nch2, K)

    epw = -(-e // NW)
    nch = -(-epw // K)
    pad1 = NW * nch * K - e
    pad1_src = jnp.full((pad1,), n, dtype=jnp.int32)
    pad1_dst = (jnp.arange(pad1, dtype=jnp.int32) % TRASH) + n
    srcs1 = jnp.concatenate([src, pad1_src]).reshape(NC, NS, nch, K)
    dsts1 = jnp.concatenate([dst, pad1_dst]).reshape(NC, NS, nch, K)

    zeros_cp = jnp.zeros((n_pad, cp), jnp.float32)
    zeros_16 = jnp.zeros((n_pad, 16), jnp.float32)
    ones_k = jnp.ones((K, 16), jnp.float32)

    prop_d = _make_prop_colsplit(n, n_pad, nch2, dh)
    prop_c = _make_prop_edgesplit(n, n_pad, nch, cp)
    deg_k = _make_deg(n, n_pad, nch)

    W4p = jnp.concatenate([W4, jnp.zeros((d, cp - c_dim), jnp.float32)], axis=1)
    b1r = b1.reshape(1, d)
    b2r = b2.reshape(1, d)
    b3r = b3.reshape(1, d)
    b4r = jnp.concatenate([b4, jnp.zeros((cp - c_dim,), jnp.float32)]).reshape(1, cp)
    batch_r = batch.reshape(1, n)

    degp = deg_k(dsts1, ones_k, zeros_16)

    def tc_split_store(t_ref, res):
        t_ref[0, :n, :] = res[:, :dh]
        t_ref[1, :n, :] = res[:, dh:]
        t_ref[0, n:, :] = jnp.zeros((TRASH, dh), jnp.float32)
        t_ref[1, n:, :] = jnp.zeros((TRASH, dh), jnp.float32)

    def tc1(x_ref, w_ref, dp_ref, t_ref, dinv_ref):
        deg = dp_ref[0, :, 0:1] + dp_ref[1, :, 0:1] + 1.0
        dinv = lax.rsqrt(deg)
        dinv_ref[...] = dinv
        res = jnp.dot(x_ref[...], w_ref[...],
                      preferred_element_type=jnp.float32) * dinv
        tc_split_store(t_ref, res)

    t1, dinv = _tc_call(
        tc1,
        (jax.ShapeDtypeStruct((NC, n_pad, dh), jnp.float32),
         jax.ShapeDtypeStruct((n, 1), jnp.float32)),
        x, W1, degp)

    p1 = prop_d(t1, srcs2, dsts2)

    def tc2(p_ref, dinv_ref, b_ref, w_ref, t2_ref, h_ref):
        dinv = dinv_ref[...]
        ap = jnp.concatenate([p_ref[0], p_ref[1]], axis=1)
        h1c = jnp.maximum(dinv * ap + b_ref[...], 0.0)
        h_ref[...] = h1c
        res = jnp.dot(h1c, w_ref[...],
                      preferred_element_type=jnp.float32) * dinv
        tc_split_store(t2_ref, res)

    t2, h1c = _tc_call(
        tc2,
        (jax.ShapeDtypeStruct((NC, n_pad, dh), jnp.float32),
         jax.ShapeDtypeStruct((n, d), jnp.float32)),
        p1, dinv, b1r, W2)

    p2 = prop_d(t2, srcs2, dsts2)

    def tc3(p_ref, h_prev_ref, dinv_ref, b_ref, w_ref, t3_ref, h_ref):
        dinv = dinv_ref[...]
        ap = jnp.concatenate([p_ref[0], p_ref[1]], axis=1)
        h2 = dinv * ap + b_ref[...]
        h2c = jnp.maximum(jnp.maximum(h2, 0.0) + h_prev_ref[...], 0.0)
        h_ref[...] = h2c
        res = jnp.dot(h2c, w_ref[...],
                      preferred_element_type=jnp.float32) * dinv
        tc_split_store(t3_ref, res)

    t3, h2c = _tc_call(
        tc3,
        (jax.ShapeDtypeStruct((NC, n_pad, dh), jnp.float32),
         jax.ShapeDtypeStruct((n, d), jnp.float32)),
        p2, h1c, dinv, b2r, W3)

    p3 = prop_d(t3, srcs2, dsts2)

    def tc4(p_ref, h_prev_ref, dinv_ref, b_ref, w_ref, t4_ref):
        dinv = dinv_ref[...]
        ap = jnp.concatenate([p_ref[0], p_ref[1]], axis=1)
        h3 = dinv * ap + b_ref[...]
        h3r = jnp.maximum(jnp.maximum(h3, 0.0) + h_prev_ref[...], 0.0)
        res = jnp.dot(h3r, w_ref[...],
                      preferred_element_type=jnp.float32) * dinv
        t4_ref[:n, :] = res
        t4_ref[n:, :] = jnp.zeros((TRASH, cp), jnp.float32)

    t4 = _tc_call(
        tc4,
        jax.ShapeDtypeStruct((n_pad, cp), jnp.float32),
        p3, h2c, dinv, b3r, W4p)

    p4 = prop_c(t4, srcs1, dsts1, zeros_cp)

    def tc5(p_ref, dinv_ref, b_ref, batch_ref, out_ref):
        h4 = dinv_ref[...] * (p_ref[0] + p_ref[1]) + b_ref[...]
        gids = lax.broadcasted_iota(jnp.int32, (g_dim, n), 0)
        onehot = (gids == batch_ref[...]).astype(jnp.float32)
        sums = jnp.dot(onehot, h4, preferred_element_type=jnp.float32)
        counts = jnp.sum(onehot, axis=1, keepdims=True)
        out_ref[...] = sums[:, :c_dim] / jnp.maximum(counts, 1.0)

    out = _tc_call(
        tc5,
        jax.ShapeDtypeStruct((g_dim, c_dim), jnp.float32),
        p4, dinv, b4r, batch_r)

    return out
